# baseline (speedup 1.0000x reference)
import functools

import jax
import jax.numpy as jnp
from jax import lax
from jax.experimental import pallas as pl
from jax.experimental.pallas import tpu as pltpu
from jax.experimental.pallas import tpu_sc as plsc

N = 10000
E = 320000
NW = 32
GEPW = E // 16
GCB = 1000
NPW = 313
NPAD = NW * NPW
SCB = 1024
BE = 3200
EG = E // BE
E2 = BE * (EG + 1)
NEG = float("-inf")

_sc_mesh = plsc.VectorSubcoreMesh(core_axis_name="c", subcore_axis_name="s")


def _wid():
    return lax.axis_index("s") * 2 + lax.axis_index("c")


def _k1_body(det, w, b, out):
    out[...] = jnp.maximum(
        jnp.dot(det[...], w[...], preferred_element_type=jnp.float32) + b[...],
        0.0,
    )


def _k1(det, w, b):
    return pl.pallas_call(
        _k1_body,
        out_shape=jax.ShapeDtypeStruct((N, 32), jnp.float32),
    )(det, w, b)


@functools.partial(
    pl.kernel,
    out_type=[
        jax.ShapeDtypeStruct((E, 32), jnp.float32),
        jax.ShapeDtypeStruct((E, 32), jnp.float32),
    ],
    mesh=_sc_mesh,
    compiler_params=pltpu.CompilerParams(use_tc_tiling_on_sc=False),
    scratch_types=[
        pltpu.VMEM((GCB + 8,), jnp.int32),
        pltpu.VMEM((GCB + 8,), jnp.int32),
        pltpu.VMEM((GCB + 8,), jnp.int32),
        pltpu.VMEM((GCB + 8, 32), jnp.float32),
        pltpu.VMEM((GCB + 8, 32), jnp.float32),
        pltpu.SemaphoreType.DMA,
        pltpu.SemaphoreType.DMA,
    ],
)
def _gather_k(h_hbm, ci_hbm, ni_hbm, cf_hbm, nf_hbm,
              ib0, ib1, cb, rb0, rb1, gs0, gs1):
    wid = _wid()
    kind = wid // 16
    estart = (wid % 16) * GEPW
    ibs, rbs, gss = (ib0, ib1), (rb0, rb1), (gs0, gs1)
    nch = GEPW // GCB
    pending = [None, None]

    for i in range(nch + 1):
        b = i % 2
        if i < nch:
            base = pl.multiple_of(estart + i * GCB, 8)

            @pl.when(kind == 0)
            def _(b=b, base=base):
                pltpu.sync_copy(ci_hbm.at[pl.ds(base, GCB + 8)], ibs[b])

            @pl.when(kind == 1)
            def _(b=b, base=base):
                pltpu.sync_copy(ni_hbm.at[pl.ds(base, GCB + 8)], ibs[b])
                pltpu.sync_copy(ci_hbm.at[pl.ds(base, GCB + 8)], cb)

                def fix(f, _):
                    o = pl.multiple_of(f * 16, 8)
                    cv = cb[pl.ds(o, 16)]
                    nv = ibs[b][pl.ds(o, 16)]
                    ibs[b][pl.ds(o, 16)] = jnp.where(
                        cv == nv, jnp.full((16,), N, jnp.int32), nv
                    )
                    return 0

                lax.fori_loop(0, (GCB + 8) // 16, fix, 0)

            pending[b] = pltpu.async_copy(h_hbm.at[ibs[b]], rbs[b], gss[b])
        if i >= 1:
            pb = (i - 1) % 2
            pbase = pl.multiple_of(estart + (i - 1) * GCB, 8)
            pending[pb].wait()

            @pl.when(kind == 0)
            def _(pb=pb, pbase=pbase):
                pltpu.sync_copy(
                    rbs[pb].at[pl.ds(0, GCB), :], cf_hbm.at[pl.ds(pbase, GCB), :]
                )

            @pl.when(kind == 1)
            def _(pb=pb, pbase=pbase):
                pltpu.sync_copy(
                    rbs[pb].at[pl.ds(0, GCB), :], nf_hbm.at[pl.ds(pbase, GCB), :]
                )


def _k2_body(p, c, n, w1, b1, w2, b2, out):
    x = (
        jnp.dot(p[...], w1[0:32, :], preferred_element_type=jnp.float32)
        + jnp.dot(c[...], w1[32:64, :], preferred_element_type=jnp.float32)
        + jnp.dot(n[...], w1[64:96, :], preferred_element_type=jnp.float32)
        + b1[...]
    )
    x = jnp.maximum(x, 0.0)
    y = jnp.dot(x, w2[...], preferred_element_type=jnp.float32) + b2[...]
    out[...] = jnp.maximum(y, 0.0)


def _k2(pair, cf, nf, w1, b1, w2, b2):
    edge_spec = pl.BlockSpec((BE, 32), lambda i: (jnp.minimum(i, EG - 1), 0))
    full = lambda s: pl.BlockSpec(s, lambda i: (0, 0))
    return pl.pallas_call(
        _k2_body,
        grid=(EG + 1,),
        in_specs=[
            edge_spec,
            edge_spec,
            edge_spec,
            full((96, 64)),
            full((1, 64)),
            full((64, 64)),
            full((1, 64)),
        ],
        out_specs=pl.BlockSpec((BE, 64), lambda i: (i, 0)),
        out_shape=jax.ShapeDtypeStruct((E2, 64), jnp.float32),
    )(pair, cf, nf, w1, b1, w2, b2)


@functools.partial(
    pl.kernel,
    out_type=jax.ShapeDtypeStruct((NPAD * 64,), jnp.float32),
    mesh=_sc_mesh,
    compiler_params=pltpu.CompilerParams(use_tc_tiling_on_sc=False),
    scratch_types=[
        pltpu.VMEM((48,), jnp.int32),
        pltpu.VMEM((SCB, 64), jnp.float32),
        pltpu.VMEM((SCB,), jnp.int32),
        pltpu.VMEM(((NPW + 1) * 64,), jnp.float32),
    ],
)
def _segmax_k(x_hbm, ci_hbm, rs_hbm, pooled_hbm, rsv, xv, iv, outv):
    wid = _wid()
    n_lo = wid * NPW
    neg = jnp.full((16,), NEG, jnp.float32)

    pltpu.sync_copy(rs_hbm, rsv)
    bounds_v = rsv[pl.ds(wid, 16)]
    e_lo = bounds_v[0]
    e_hi = bounds_v[1]

    def init(i, _):
        outv[pl.ds(pl.multiple_of(i * 16, 8), 16)] = neg
        return 0

    lax.fori_loop(0, (NPW + 1) * 4, init, 0)

    a_lo = pl.multiple_of((e_lo // 8) * 8, 8)
    nch = (e_hi - a_lo + SCB - 1) // SCB
    lane = jnp.arange(16, dtype=jnp.int32)

    def chunk(c, carry):
        base = pl.multiple_of(a_lo + c * SCB, 8)
        pltpu.sync_copy(x_hbm.at[pl.ds(base, SCB), :], xv)
        pltpu.sync_copy(ci_hbm.at[pl.ds(base, SCB)], iv)

        def edge16(eb, carry):
            prev, a0, a1, a2, a3 = carry
            e0 = pl.multiple_of(eb * 16, 8)
            idv = iv[pl.ds(e0, 16)]
            gv = base + e0 + lane
            valid = jnp.logical_and(gv >= e_lo, gv < e_hi)
            rowv = jnp.where(valid, idv - n_lo, NPW)
            for j in range(16):
                row = rowv[j]
                changed = row != prev
                x0 = xv[e0 + j, pl.ds(0, 16)]
                x1 = xv[e0 + j, pl.ds(16, 16)]
                x2 = xv[e0 + j, pl.ds(32, 16)]
                x3 = xv[e0 + j, pl.ds(48, 16)]
                a0 = jnp.maximum(jnp.where(changed, neg, a0), x0)
                a1 = jnp.maximum(jnp.where(changed, neg, a1), x1)
                a2 = jnp.maximum(jnp.where(changed, neg, a2), x2)
                a3 = jnp.maximum(jnp.where(changed, neg, a3), x3)
                off = pl.multiple_of(row * 64, 8)
                outv[pl.ds(off, 16)] = a0
                outv[pl.ds(off + 16, 16)] = a1
                outv[pl.ds(off + 32, 16)] = a2
                outv[pl.ds(off + 48, 16)] = a3
                prev = row
            return (prev, a0, a1, a2, a3)

        return lax.fori_loop(0, SCB // 16, edge16, carry)

    carry0 = (jnp.int32(NPW), neg, neg, neg, neg)
    lax.fori_loop(0, nch, chunk, carry0)

    pltpu.sync_copy(
        outv.at[pl.ds(0, NPW * 64)],
        pooled_hbm.at[pl.ds(wid * (NPW * 64), NPW * 64)],
    )


def _k3_body(pooled, det, wm1, bm1, wm2, bm2, wo, bo, out):
    t = jnp.maximum(
        jnp.dot(pooled[...], wm1[...], preferred_element_type=jnp.float32) + bm1[...],
        0.0,
    )
    t = jnp.maximum(
        jnp.dot(t, wm2[...], preferred_element_type=jnp.float32) + bm2[...], 0.0
    )
    o = jnp.dot(t, wo[...], preferred_element_type=jnp.float32) + bo[...]
    out[...] = jnp.maximum(det[...] + o, 0.0)


def _k3(pooled, det, wm1, bm1, wm2, bm2, wo, bo):
    return pl.pallas_call(
        _k3_body,
        out_shape=jax.ShapeDtypeStruct((N, 128), jnp.float32),
    )(pooled, det, wm1, bm1, wm2, bm2, wo, bo)


def kernel(detFeatures, cIdxs, nIdxs, pairFeatures,
           W_fc1, b_fc1, W_pw1, b_pw1, W_pw2, b_pw2,
           W_pm1, b_pm1, W_pm2, b_pm2, W_out, b_out):
    ci = cIdxs.astype(jnp.int32)
    ni = nIdxs.astype(jnp.int32)

    h = _k1(detFeatures, W_fc1, b_fc1.reshape(1, 32))
    h_ext = jnp.concatenate([h, jnp.zeros((8, 32), jnp.float32)], axis=0)

    ci_pad = jnp.concatenate([ci, jnp.zeros((E2 - E,), jnp.int32)])
    ni_pad = jnp.concatenate([ni, jnp.zeros((E2 - E,), jnp.int32)])
    cf, nf = _gather_k(h_ext, ci_pad, ni_pad)

    x = _k2(pairFeatures, cf, nf, W_pw1, b_pw1.reshape(1, 64),
            W_pw2, b_pw2.reshape(1, 64))

    bounds = jnp.minimum(jnp.arange(33, dtype=jnp.int32) * NPW, N)
    rs = jnp.searchsorted(ci, bounds).astype(jnp.int32)
    rs = jnp.concatenate([rs, jnp.zeros((15,), jnp.int32)])

    pooled = _segmax_k(x, ci_pad, rs)
    pooled = pooled.reshape(NPAD, 64)[:N]

    return _k3(pooled, detFeatures, W_pm1, b_pm1.reshape(1, 64),
               W_pm2, b_pm2.reshape(1, 64), W_out, b_out.reshape(1, 128))


# speedup vs baseline: 1.0064x; 1.0064x over previous
import functools

import jax
import jax.numpy as jnp
from jax import lax
from jax.experimental import pallas as pl
from jax.experimental.pallas import tpu as pltpu
from jax.experimental.pallas import tpu_sc as plsc

N = 10000
E = 320000
NW = 32
EPW = E // NW
GCB = 2000
NPW = 313
NPAD = NW * NPW
SCB = 1024
BE = 3200
EG = E // BE
E2 = BE * (EG + 1)
NEG = float("-inf")

_sc_mesh = plsc.VectorSubcoreMesh(core_axis_name="c", subcore_axis_name="s")


def _wid():
    return lax.axis_index("s") * 2 + lax.axis_index("c")


def _k1_body(det, w, b, out):
    out[...] = jnp.maximum(
        jnp.dot(det[...], w[...], preferred_element_type=jnp.float32) + b[...],
        0.0,
    )


def _k1(det, w, b):
    return pl.pallas_call(
        _k1_body,
        out_shape=jax.ShapeDtypeStruct((N, 32), jnp.float32),
    )(det, w, b)


@functools.partial(
    pl.kernel,
    out_type=[
        jax.ShapeDtypeStruct((E, 32), jnp.float32),
        jax.ShapeDtypeStruct((E, 32), jnp.float32),
    ],
    mesh=_sc_mesh,
    compiler_params=pltpu.CompilerParams(use_tc_tiling_on_sc=False),
    scratch_types=[
        pltpu.VMEM((GCB,), jnp.int32),
        pltpu.VMEM((GCB,), jnp.int32),
        pltpu.VMEM((GCB, 32), jnp.float32),
        pltpu.SemaphoreType.DMA,
    ],
)
def _gather_k(h_hbm, ci_hbm, ni_hbm, cf_hbm, nf_hbm, civ, niv, rows, sem):
    wid = _wid()

    def chunk(c, _):
        base = pl.multiple_of(wid * EPW + c * GCB, 8)
        pltpu.sync_copy(ci_hbm.at[pl.ds(base, GCB)], civ)
        pltpu.sync_copy(ni_hbm.at[pl.ds(base, GCB)], niv)

        def fix(i, _):
            o = pl.multiple_of(i * 16, 8)
            cv = civ[pl.ds(o, 16)]
            nv = niv[pl.ds(o, 16)]
            niv[pl.ds(o, 16)] = jnp.where(
                cv == nv, jnp.full((16,), N, jnp.int32), nv
            )
            return 0

        lax.fori_loop(0, GCB // 16, fix, 0)
        pltpu.async_copy(h_hbm.at[civ], rows, sem).wait()
        pltpu.sync_copy(rows, cf_hbm.at[pl.ds(base, GCB), :])
        pltpu.async_copy(h_hbm.at[niv], rows, sem).wait()
        pltpu.sync_copy(rows, nf_hbm.at[pl.ds(base, GCB), :])
        return 0

    lax.fori_loop(0, EPW // GCB, chunk, 0)


def _k2_body(p, c, n, w1, b1, w2, b2, out):
    x = (
        jnp.dot(p[...], w1[0:32, :], preferred_element_type=jnp.float32)
        + jnp.dot(c[...], w1[32:64, :], preferred_element_type=jnp.float32)
        + jnp.dot(n[...], w1[64:96, :], preferred_element_type=jnp.float32)
        + b1[...]
    )
    x = jnp.maximum(x, 0.0)
    y = jnp.dot(x, w2[...], preferred_element_type=jnp.float32) + b2[...]
    out[...] = jnp.maximum(y, 0.0)


def _k2(pair, cf, nf, w1, b1, w2, b2):
    edge_spec = pl.BlockSpec((BE, 32), lambda i: (jnp.minimum(i, EG - 1), 0))
    full = lambda s: pl.BlockSpec(s, lambda i: (0, 0))
    return pl.pallas_call(
        _k2_body,
        grid=(EG + 1,),
        in_specs=[
            edge_spec,
            edge_spec,
            edge_spec,
            full((96, 64)),
            full((1, 64)),
            full((64, 64)),
            full((1, 64)),
        ],
        out_specs=pl.BlockSpec((BE, 64), lambda i: (i, 0)),
        out_shape=jax.ShapeDtypeStruct((E2, 64), jnp.float32),
    )(pair, cf, nf, w1, b1, w2, b2)


@functools.partial(
    pl.kernel,
    out_type=jax.ShapeDtypeStruct((NPAD * 64,), jnp.float32),
    mesh=_sc_mesh,
    compiler_params=pltpu.CompilerParams(use_tc_tiling_on_sc=False),
    scratch_types=[
        pltpu.VMEM((48,), jnp.int32),
        pltpu.VMEM((SCB, 64), jnp.float32),
        pltpu.VMEM((SCB,), jnp.int32),
        pltpu.VMEM(((NPW + 1) * 64,), jnp.float32),
    ],
)
def _segmax_k(x_hbm, ci_hbm, rs_hbm, pooled_hbm, rsv, xv, iv, outv):
    wid = _wid()
    n_lo = wid * NPW
    neg = jnp.full((16,), NEG, jnp.float32)

    pltpu.sync_copy(rs_hbm, rsv)
    bounds_v = rsv[pl.ds(wid, 16)]
    e_lo = bounds_v[0]
    e_hi = bounds_v[1]

    def init(i, _):
        outv[pl.ds(pl.multiple_of(i * 16, 8), 16)] = neg
        return 0

    lax.fori_loop(0, (NPW + 1) * 4, init, 0)

    a_lo = pl.multiple_of((e_lo // 8) * 8, 8)
    nch = (e_hi - a_lo + SCB - 1) // SCB
    lane = jnp.arange(16, dtype=jnp.int32)

    def chunk(c, carry):
        base = pl.multiple_of(a_lo + c * SCB, 8)
        pltpu.sync_copy(x_hbm.at[pl.ds(base, SCB), :], xv)
        pltpu.sync_copy(ci_hbm.at[pl.ds(base, SCB)], iv)

        def edge16(eb, carry):
            prev, a0, a1, a2, a3 = carry
            e0 = pl.multiple_of(eb * 16, 8)
            idv = iv[pl.ds(e0, 16)]
            gv = base + e0 + lane
            valid = jnp.logical_and(gv >= e_lo, gv < e_hi)
            rowv = jnp.where(valid, idv - n_lo, NPW)
            for j in range(16):
                row = rowv[j]
                changed = row != prev
                x0 = xv[e0 + j, pl.ds(0, 16)]
                x1 = xv[e0 + j, pl.ds(16, 16)]
                x2 = xv[e0 + j, pl.ds(32, 16)]
                x3 = xv[e0 + j, pl.ds(48, 16)]
                a0 = jnp.maximum(jnp.where(changed, neg, a0), x0)
                a1 = jnp.maximum(jnp.where(changed, neg, a1), x1)
                a2 = jnp.maximum(jnp.where(changed, neg, a2), x2)
                a3 = jnp.maximum(jnp.where(changed, neg, a3), x3)
                off = pl.multiple_of(row * 64, 8)
                outv[pl.ds(off, 16)] = a0
                outv[pl.ds(off + 16, 16)] = a1
                outv[pl.ds(off + 32, 16)] = a2
                outv[pl.ds(off + 48, 16)] = a3
                prev = row
            return (prev, a0, a1, a2, a3)

        return lax.fori_loop(0, SCB // 16, edge16, carry)

    carry0 = (jnp.int32(NPW), neg, neg, neg, neg)
    lax.fori_loop(0, nch, chunk, carry0)

    pltpu.sync_copy(
        outv.at[pl.ds(0, NPW * 64)],
        pooled_hbm.at[pl.ds(wid * (NPW * 64), NPW * 64)],
    )


def _k3_body(pooled, det, wm1, bm1, wm2, bm2, wo, bo, out):
    t = jnp.maximum(
        jnp.dot(pooled[...], wm1[...], preferred_element_type=jnp.float32) + bm1[...],
        0.0,
    )
    t = jnp.maximum(
        jnp.dot(t, wm2[...], preferred_element_type=jnp.float32) + bm2[...], 0.0
    )
    o = jnp.dot(t, wo[...], preferred_element_type=jnp.float32) + bo[...]
    out[...] = jnp.maximum(det[...] + o, 0.0)


def _k3(pooled, det, wm1, bm1, wm2, bm2, wo, bo):
    return pl.pallas_call(
        _k3_body,
        out_shape=jax.ShapeDtypeStruct((N, 128), jnp.float32),
    )(pooled, det, wm1, bm1, wm2, bm2, wo, bo)


def kernel(detFeatures, cIdxs, nIdxs, pairFeatures,
           W_fc1, b_fc1, W_pw1, b_pw1, W_pw2, b_pw2,
           W_pm1, b_pm1, W_pm2, b_pm2, W_out, b_out):
    ci = cIdxs.astype(jnp.int32)
    ni = nIdxs.astype(jnp.int32)

    h = _k1(detFeatures, W_fc1, b_fc1.reshape(1, 32))
    h_ext = jnp.concatenate([h, jnp.zeros((8, 32), jnp.float32)], axis=0)

    ci_pad = jnp.concatenate([ci, jnp.zeros((E2 - E,), jnp.int32)])
    ni_pad = jnp.concatenate([ni, jnp.zeros((E2 - E,), jnp.int32)])
    cf, nf = _gather_k(h_ext, ci_pad, ni_pad)

    x = _k2(pairFeatures, cf, nf, W_pw1, b_pw1.reshape(1, 64),
            W_pw2, b_pw2.reshape(1, 64))

    bounds = jnp.minimum(jnp.arange(33, dtype=jnp.int32) * NPW, N)
    rs = jnp.searchsorted(ci, bounds).astype(jnp.int32)
    rs = jnp.concatenate([rs, jnp.zeros((15,), jnp.int32)])

    pooled = _segmax_k(x, ci_pad, rs)
    pooled = pooled.reshape(NPAD, 64)[:N]

    return _k3(pooled, detFeatures, W_pm1, b_pm1.reshape(1, 64),
               W_pm2, b_pm2.reshape(1, 64), W_out, b_out.reshape(1, 128))


# speedup vs baseline: 1.0586x; 1.0519x over previous
import functools

import jax
import jax.numpy as jnp
from jax import lax
from jax.experimental import pallas as pl
from jax.experimental.pallas import tpu as pltpu
from jax.experimental.pallas import tpu_sc as plsc

N = 10000
E = 320000
NW = 32
EPW = E // NW
GCB = 2000
NPW = 313
NPAD = NW * NPW
SCB = 1024
BE = 3200
EG = E // BE
E2 = BE * (EG + 1)
NEG = float("-inf")

_sc_mesh = plsc.VectorSubcoreMesh(core_axis_name="c", subcore_axis_name="s")


def _wid():
    return lax.axis_index("s") * 2 + lax.axis_index("c")


def _k1_body(det, w, b, out):
    out[...] = jnp.maximum(
        jnp.dot(det[...], w[...], preferred_element_type=jnp.float32) + b[...],
        0.0,
    )


def _k1(det, w, b):
    return pl.pallas_call(
        _k1_body,
        out_shape=jax.ShapeDtypeStruct((N, 32), jnp.float32),
    )(det, w, b)


@functools.partial(
    pl.kernel,
    out_type=[
        jax.ShapeDtypeStruct((E, 32), jnp.float32),
        jax.ShapeDtypeStruct((E, 32), jnp.float32),
    ],
    mesh=_sc_mesh,
    compiler_params=pltpu.CompilerParams(use_tc_tiling_on_sc=False),
    scratch_types=[
        pltpu.VMEM((GCB,), jnp.int32),
        pltpu.VMEM((GCB,), jnp.int32),
        pltpu.VMEM((GCB, 32), jnp.float32),
        pltpu.SemaphoreType.DMA,
    ],
)
def _gather_k(h_hbm, ci_hbm, ni_hbm, cf_hbm, nf_hbm, civ, niv, rows, sem):
    wid = _wid()

    def chunk(c, _):
        base = pl.multiple_of(wid * EPW + c * GCB, 8)
        pltpu.sync_copy(ci_hbm.at[pl.ds(base, GCB)], civ)
        pltpu.sync_copy(ni_hbm.at[pl.ds(base, GCB)], niv)

        def fix(i, _):
            o = pl.multiple_of(i * 16, 8)
            cv = civ[pl.ds(o, 16)]
            nv = niv[pl.ds(o, 16)]
            niv[pl.ds(o, 16)] = jnp.where(
                cv == nv, jnp.full((16,), N, jnp.int32), nv
            )
            return 0

        lax.fori_loop(0, GCB // 16, fix, 0)
        pltpu.async_copy(h_hbm.at[civ], rows, sem).wait()
        pltpu.sync_copy(rows, cf_hbm.at[pl.ds(base, GCB), :])
        pltpu.async_copy(h_hbm.at[niv], rows, sem).wait()
        pltpu.sync_copy(rows, nf_hbm.at[pl.ds(base, GCB), :])
        return 0

    lax.fori_loop(0, EPW // GCB, chunk, 0)


def _k2_body(p, c, n, w1, b1, w2, b2, out):
    x = (
        jnp.dot(p[...], w1[0:32, :], preferred_element_type=jnp.float32)
        + jnp.dot(c[...], w1[32:64, :], preferred_element_type=jnp.float32)
        + jnp.dot(n[...], w1[64:96, :], preferred_element_type=jnp.float32)
        + b1[...]
    )
    x = jnp.maximum(x, 0.0)
    y = jnp.dot(x, w2[...], preferred_element_type=jnp.float32) + b2[...]
    out[...] = jnp.maximum(y, 0.0)


def _k2(pair, cf, nf, w1, b1, w2, b2):
    edge_spec = pl.BlockSpec((BE, 32), lambda i: (jnp.minimum(i, EG - 1), 0))
    full = lambda s: pl.BlockSpec(s, lambda i: (0, 0))
    return pl.pallas_call(
        _k2_body,
        grid=(EG + 1,),
        in_specs=[
            edge_spec,
            edge_spec,
            edge_spec,
            full((96, 64)),
            full((1, 64)),
            full((64, 64)),
            full((1, 64)),
        ],
        out_specs=pl.BlockSpec((BE, 64), lambda i: (i, 0)),
        out_shape=jax.ShapeDtypeStruct((E2, 64), jnp.float32),
    )(pair, cf, nf, w1, b1, w2, b2)


@functools.partial(
    pl.kernel,
    out_type=jax.ShapeDtypeStruct((NPAD * 64,), jnp.float32),
    mesh=_sc_mesh,
    compiler_params=pltpu.CompilerParams(use_tc_tiling_on_sc=False),
    scratch_types=[
        pltpu.VMEM((48,), jnp.int32),
        pltpu.VMEM((SCB, 64), jnp.float32),
        pltpu.VMEM((SCB,), jnp.int32),
        pltpu.VMEM(((NPW + 1) * 64,), jnp.float32),
    ],
)
def _segmax_k(x_hbm, ci_hbm, rs_hbm, pooled_hbm, rsv, xv, iv, outv):
    wid = _wid()
    n_lo = wid * NPW
    neg = jnp.full((16,), NEG, jnp.float32)

    pltpu.sync_copy(rs_hbm, rsv)
    bounds_v = rsv[pl.ds(wid, 16)]
    e_lo = bounds_v[0]
    e_hi = bounds_v[1]

    def init(i, _):
        outv[pl.ds(pl.multiple_of(i * 16, 8), 16)] = neg
        return 0

    lax.fori_loop(0, (NPW + 1) * 4, init, 0)

    a_lo = pl.multiple_of((e_lo // 8) * 8, 8)
    nch = (e_hi - a_lo + SCB - 1) // SCB
    lane = jnp.arange(16, dtype=jnp.int32)

    def chunk(c, carry):
        base = pl.multiple_of(a_lo + c * SCB, 8)
        pltpu.sync_copy(x_hbm.at[pl.ds(base, SCB), :], xv)
        pltpu.sync_copy(ci_hbm.at[pl.ds(base, SCB)], iv)

        def edge16(eb, carry):
            prev, a0, a1, a2, a3 = carry
            e0 = pl.multiple_of(eb * 16, 8)
            idv = iv[pl.ds(e0, 16)]
            gv = base + e0 + lane
            valid = jnp.logical_and(gv >= e_lo, gv < e_hi)
            rowv = jnp.where(valid, idv - n_lo, NPW)
            for j in range(16):
                row = rowv[j]
                changed = row != prev
                x0 = xv[e0 + j, pl.ds(0, 16)]
                x1 = xv[e0 + j, pl.ds(16, 16)]
                x2 = xv[e0 + j, pl.ds(32, 16)]
                x3 = xv[e0 + j, pl.ds(48, 16)]
                a0 = jnp.maximum(jnp.where(changed, neg, a0), x0)
                a1 = jnp.maximum(jnp.where(changed, neg, a1), x1)
                a2 = jnp.maximum(jnp.where(changed, neg, a2), x2)
                a3 = jnp.maximum(jnp.where(changed, neg, a3), x3)
                off = pl.multiple_of(row * 64, 8)
                outv[pl.ds(off, 16)] = a0
                outv[pl.ds(off + 16, 16)] = a1
                outv[pl.ds(off + 32, 16)] = a2
                outv[pl.ds(off + 48, 16)] = a3
                prev = row
            return (prev, a0, a1, a2, a3)

        return lax.fori_loop(0, SCB // 16, edge16, carry)

    carry0 = (jnp.int32(NPW), neg, neg, neg, neg)
    lax.fori_loop(0, nch, chunk, carry0)

    pltpu.sync_copy(
        outv.at[pl.ds(0, NPW * 64)],
        pooled_hbm.at[pl.ds(wid * (NPW * 64), NPW * 64)],
    )


def _k3_body(pooled, det, wm1, bm1, wm2, bm2, wo, bo, out):
    t = jnp.maximum(
        jnp.dot(pooled[...], wm1[...], preferred_element_type=jnp.float32) + bm1[...],
        0.0,
    )
    t = jnp.maximum(
        jnp.dot(t, wm2[...], preferred_element_type=jnp.float32) + bm2[...], 0.0
    )
    o = jnp.dot(t, wo[...], preferred_element_type=jnp.float32) + bo[...]
    out[...] = jnp.maximum(det[...] + o, 0.0)


def _k3(pooled, det, wm1, bm1, wm2, bm2, wo, bo):
    return pl.pallas_call(
        _k3_body,
        out_shape=jax.ShapeDtypeStruct((N, 128), jnp.float32),
    )(pooled, det, wm1, bm1, wm2, bm2, wo, bo)


def kernel(detFeatures, cIdxs, nIdxs, pairFeatures,
           W_fc1, b_fc1, W_pw1, b_pw1, W_pw2, b_pw2,
           W_pm1, b_pm1, W_pm2, b_pm2, W_out, b_out):
    ci = cIdxs.astype(jnp.int32)
    ni = nIdxs.astype(jnp.int32)

    h = _k1(detFeatures, W_fc1, b_fc1.reshape(1, 32))
    h_ext = jnp.concatenate([h, jnp.zeros((8, 32), jnp.float32)], axis=0)

    cf, nf = _gather_k(h_ext, ci, ni)

    x = _k2(pairFeatures, cf, nf, W_pw1, b_pw1.reshape(1, 64),
            W_pw2, b_pw2.reshape(1, 64))

    bounds = jnp.minimum(jnp.arange(33, dtype=jnp.int32) * NPW, N)
    rs = jnp.searchsorted(ci, bounds).astype(jnp.int32)
    rs = jnp.concatenate([rs, jnp.zeros((15,), jnp.int32)])
    ci_pad = jnp.concatenate([ci, jnp.zeros((E2 - E,), jnp.int32)])

    pooled = _segmax_k(x, ci_pad, rs)
    pooled = pooled.reshape(NPAD, 64)[:N]

    return _k3(pooled, detFeatures, W_pm1, b_pm1.reshape(1, 64),
               W_pm2, b_pm2.reshape(1, 64), W_out, b_out.reshape(1, 128))
